# Initial kernel scaffold; baseline (speedup 1.0000x reference)
import jax
import jax.numpy as jnp
from jax.experimental import pallas as pl


def kernel(x, edge_index, W1, b1, W2, b2, W_ih_f, W_hh_f, b_ih_f, b_hh_f, W_ih_b, W_hh_b, b_ih_b, b_hh_b, W_att, b_att, W3, b3):
    raise NotImplementedError("write your pallas kernel here")


# speedup vs baseline: 14.1948x; 14.1948x over previous
import functools

import jax
import jax.numpy as jnp
from jax import lax
from jax.experimental import pallas as pl
from jax.experimental.pallas import tpu as pltpu
from jax.experimental.pallas import tpu_sc as plsc

NN = 10000
NP = 10240
EE = 320000
CH = 128
NCHUNK = EE // CH
NC = 2
NS = 16
NW = NC * NS
RPS = NP // NS
RWB = 128
MAXT = (NCHUNK + NW - 1) // NW

RB = 2000


def _sc_mesh():
    return plsc.VectorSubcoreMesh(core_axis_name="c", subcore_axis_name="s",
                                  num_cores=NC, num_subcores=NS)


def _degree_body(e_hbm, ones_hbm, zer_hbm, out_hbm, ibuf, obuf, wbuf, acc):
    c = lax.axis_index("c")
    s = lax.axis_index("s")
    w = c * NS + s
    base = s * RPS
    pltpu.sync_copy(zer_hbm, wbuf)
    pltpu.sync_copy(ones_hbm, obuf)
    for k in range(RPS // RWB):
        pltpu.sync_copy(wbuf, acc.at[pl.ds(base + k * RWB, RWB)])
    plsc.subcore_barrier()

    def body(t, carry):
        j = w + NW * t

        @pl.when(j < NCHUNK)
        def _():
            pltpu.sync_copy(e_hbm.at[j], ibuf)
            pltpu.sync_copy(obuf, acc.at[ibuf.at[1]], add=True)

        return carry

    lax.fori_loop(0, MAXT, body, 0)
    plsc.subcore_barrier()
    row0 = c * NP + base
    for k in range(RPS // RWB):
        pltpu.sync_copy(acc.at[pl.ds(base + k * RWB, RWB)], wbuf)
        pltpu.sync_copy(wbuf, out_hbm.at[pl.ds(row0 + k * RWB, RWB)])


def _sc_degree(echunks, ones128, zeros128):
    return pl.kernel(
        _degree_body,
        out_type=jax.ShapeDtypeStruct((2 * NP, 128), jnp.float32),
        mesh=_sc_mesh(),
        scratch_types=[
            pltpu.VMEM((2, CH), jnp.int32),
            pltpu.VMEM((CH, 128), jnp.float32),
            pltpu.VMEM((RWB, 128), jnp.float32),
            pltpu.VMEM_SHARED((NP, 128), jnp.float32),
        ],
    )(echunks, ones128, zeros128)


def _prop_body(e_hbm, tab_hbm, zer_hbm, out_hbm, ibuf, gbuf, wbuf, acc):
    c = lax.axis_index("c")
    s = lax.axis_index("s")
    w = c * NS + s
    base = s * RPS
    pltpu.sync_copy(zer_hbm, wbuf)
    for k in range(RPS // RWB):
        pltpu.sync_copy(wbuf, acc.at[pl.ds(base + k * RWB, RWB)])
    plsc.subcore_barrier()

    def body(t, carry):
        j = w + NW * t

        @pl.when(j < NCHUNK)
        def _():
            pltpu.sync_copy(e_hbm.at[j], ibuf)
            pltpu.sync_copy(tab_hbm.at[ibuf.at[0]], gbuf)
            pltpu.sync_copy(gbuf, acc.at[ibuf.at[1]], add=True)

        return carry

    lax.fori_loop(0, MAXT, body, 0)
    plsc.subcore_barrier()
    row0 = c * NP + base
    for k in range(RPS // RWB):
        pltpu.sync_copy(acc.at[pl.ds(base + k * RWB, RWB)], wbuf)
        pltpu.sync_copy(wbuf, out_hbm.at[pl.ds(row0 + k * RWB, RWB)])


def _sc_prop(echunks, table, zeros128):
    return pl.kernel(
        _prop_body,
        out_type=jax.ShapeDtypeStruct((2 * NP, 128), jnp.float32),
        mesh=_sc_mesh(),
        scratch_types=[
            pltpu.VMEM((2, CH), jnp.int32),
            pltpu.VMEM((CH, 128), jnp.float32),
            pltpu.VMEM((RWB, 128), jnp.float32),
            pltpu.VMEM_SHARED((NP, 128), jnp.float32),
        ],
    )(echunks, table, zeros128)


def _dinv(dega, degb):
    return lax.rsqrt(dega[:, 0:1] + degb[:, 0:1] + 1.0)


def _b_body(x, w1, dega, degb, h0p):
    h0p[...] = jnp.dot(x[...], w1[...],
                       preferred_element_type=jnp.float32) * _dinv(dega[...], degb[...])


def _tc_b(x, W1, dega, degb):
    grid = (NN // RB,)
    return pl.pallas_call(
        _b_body,
        grid=grid,
        in_specs=[
            pl.BlockSpec((RB, 128), lambda i: (i, 0)),
            pl.BlockSpec((128, 128), lambda i: (0, 0)),
            pl.BlockSpec((RB, 16), lambda i: (i, 0)),
            pl.BlockSpec((RB, 16), lambda i: (i, 0)),
        ],
        out_specs=pl.BlockSpec((RB, 128), lambda i: (i, 0)),
        out_shape=jax.ShapeDtypeStruct((NN, 128), jnp.float32),
    )(x, W1, dega, degb)


def _d_body(s0a, s0b, h0p, dega, degb, w2, b1, x1, h1p):
    di = _dinv(dega[...], degb[...])
    xx = jnp.maximum((s0a[...] + s0b[...] + h0p[...]) * di + b1[...], 0.0)
    x1[...] = xx
    h1p[...] = jnp.dot(xx, w2[...], preferred_element_type=jnp.float32) * di


def _tc_d(s0a, s0b, h0p, dega, degb, W2, b1r):
    grid = (NN // RB,)
    blk = pl.BlockSpec((RB, 128), lambda i: (i, 0))
    deg = pl.BlockSpec((RB, 16), lambda i: (i, 0))
    return pl.pallas_call(
        _d_body,
        grid=grid,
        in_specs=[blk, blk, blk, deg, deg,
                  pl.BlockSpec((128, 128), lambda i: (0, 0)),
                  pl.BlockSpec((1, 128), lambda i: (0, 0))],
        out_specs=[blk, blk],
        out_shape=[jax.ShapeDtypeStruct((NN, 128), jnp.float32),
                   jax.ShapeDtypeStruct((NN, 128), jnp.float32)],
    )(s0a, s0b, h0p, dega, degb, W2, b1r)


def _sig(v):
    return jax.nn.sigmoid(v)


def _f_body(s1a, s1b, h1p, x1r, dega, degb, b2, wihf, whhf, bf, wihb, whhb,
            bb, waf, wab, xjkp):
    di = _dinv(dega[...], degb[...])
    x1 = x1r[...]
    x2 = jnp.maximum((s1a[...] + s1b[...] + h1p[...]) * di + b2[...], 0.0)

    def mm(a, b):
        return jnp.dot(a, b, preferred_element_type=jnp.float32)

    g = mm(x1, wihf[...]) + bf[...]
    cf0 = _sig(g[:, 0:128]) * jnp.tanh(g[:, 256:384])
    hf0 = _sig(g[:, 384:512]) * jnp.tanh(cf0)
    g = mm(x2, wihf[...]) + mm(hf0, whhf[...]) + bf[...]
    cf1 = _sig(g[:, 128:256]) * cf0 + _sig(g[:, 0:128]) * jnp.tanh(g[:, 256:384])
    hf1 = _sig(g[:, 384:512]) * jnp.tanh(cf1)
    g = mm(x2, wihb[...]) + bb[...]
    cb1 = _sig(g[:, 0:128]) * jnp.tanh(g[:, 256:384])
    hb1 = _sig(g[:, 384:512]) * jnp.tanh(cb1)
    g = mm(x1, wihb[...]) + mm(hb1, whhb[...]) + bb[...]
    cb0 = _sig(g[:, 128:256]) * cb1 + _sig(g[:, 0:128]) * jnp.tanh(g[:, 256:384])
    hb0 = _sig(g[:, 384:512]) * jnp.tanh(cb0)
    sc0 = (jnp.sum(hf0 * waf[...], axis=1, keepdims=True)
           + jnp.sum(hb0 * wab[...], axis=1, keepdims=True))
    sc1 = (jnp.sum(hf1 * waf[...], axis=1, keepdims=True)
           + jnp.sum(hb1 * wab[...], axis=1, keepdims=True))
    m = jnp.maximum(sc0, sc1)
    e0 = jnp.exp(sc0 - m)
    e1 = jnp.exp(sc1 - m)
    inv = 1.0 / (e0 + e1)
    xjkp[...] = (e0 * inv * x1 + e1 * inv * x2) * di


def _tc_f(s1a, s1b, h1p, x1, dega, degb, b2r, wihf, whhf, bfr, wihb, whhb,
          bbr, wafr, wabr):
    grid = (NN // RB,)
    blk = pl.BlockSpec((RB, 128), lambda i: (i, 0))
    deg = pl.BlockSpec((RB, 16), lambda i: (i, 0))
    wgt = pl.BlockSpec((128, 512), lambda i: (0, 0))
    vec = pl.BlockSpec((1, 512), lambda i: (0, 0))
    row = pl.BlockSpec((1, 128), lambda i: (0, 0))
    return pl.pallas_call(
        _f_body,
        grid=grid,
        in_specs=[blk, blk, blk, blk, deg, deg, row,
                  wgt, wgt, vec, wgt, wgt, vec, row, row],
        out_specs=blk,
        out_shape=jax.ShapeDtypeStruct((NN, 128), jnp.float32),
    )(s1a, s1b, h1p, x1, dega, degb, b2r, wihf, whhf, bfr, wihb, whhb, bbr,
      wafr, wabr)


def _h_body(s2a, s2b, xjkp, dega, degb, w3, b3, out):
    di = _dinv(dega[...], degb[...])
    xp = (s2a[...] + s2b[...] + xjkp[...]) * di
    lo = jnp.dot(xp, w3[...], preferred_element_type=jnp.float32) + b3[...]
    m = jnp.max(lo, axis=1, keepdims=True)
    e = jnp.exp(lo - m)
    out[...] = lo - m - jnp.log(jnp.sum(e, axis=1, keepdims=True))


def _tc_h(s2a, s2b, xjkp, dega, degb, W3, b3r):
    grid = (NN // RB,)
    blk = pl.BlockSpec((RB, 128), lambda i: (i, 0))
    deg = pl.BlockSpec((RB, 16), lambda i: (i, 0))
    return pl.pallas_call(
        _h_body,
        grid=grid,
        in_specs=[blk, blk, blk, deg, deg,
                  pl.BlockSpec((128, 40), lambda i: (0, 0)),
                  pl.BlockSpec((1, 40), lambda i: (0, 0))],
        out_specs=pl.BlockSpec((RB, 40), lambda i: (i, 0)),
        out_shape=jax.ShapeDtypeStruct((NN, 40), jnp.float32),
    )(s2a, s2b, xjkp, dega, degb, W3, b3r)


def kernel(x, edge_index, W1, b1, W2, b2, W_ih_f, W_hh_f, b_ih_f, b_hh_f,
           W_ih_b, W_hh_b, b_ih_b, b_hh_b, W_att, b_att, W3, b3):
    ei = edge_index.astype(jnp.int32)
    echunks = ei.reshape(2, NCHUNK, CH).transpose(1, 0, 2)
    ones128 = jnp.ones((CH, 128), jnp.float32)
    zeros128 = jnp.zeros((RWB, 128), jnp.float32)

    deg2 = _sc_degree(echunks, ones128, zeros128)
    dega, degb = deg2[:NN, :16], deg2[NP:NP + NN, :16]

    h0p = _tc_b(x, W1, dega, degb)
    s0 = _sc_prop(echunks, h0p, zeros128)
    x1, h1p = _tc_d(s0[:NN], s0[NP:NP + NN], h0p, dega, degb, W2, b1.reshape(1, 128))
    s1 = _sc_prop(echunks, h1p, zeros128)
    xjkp = _tc_f(
        s1[:NN], s1[NP:NP + NN], h1p, x1, dega, degb, b2.reshape(1, 128),
        W_ih_f.T, W_hh_f.T, (b_ih_f + b_hh_f).reshape(1, 512),
        W_ih_b.T, W_hh_b.T, (b_ih_b + b_hh_b).reshape(1, 512),
        W_att[:128, 0].reshape(1, 128), W_att[128:, 0].reshape(1, 128))
    s2 = _sc_prop(echunks, xjkp, zeros128)
    return _tc_h(s2[:NN], s2[NP:NP + NN], xjkp, dega, degb, W3, b3.reshape(1, 40))
